# baseline (speedup 1.0000x reference)
import functools

import jax
import jax.numpy as jnp
from jax.experimental import pallas as pl
from jax.experimental.pallas import tpu as pltpu

_BETA = 1.0 / 9.0
_LANES = 640


def _partials_kernel(p_ref, t_ref, out_ref, *, beta, half_over_beta, half_beta):
    cols = jax.lax.broadcasted_iota(jnp.int32, (1, _LANES), 1)
    mask = ((cols % 5) < 2).astype(jnp.float32)
    diff = jnp.abs(p_ref[...] - t_ref[...])
    elem = jnp.where(diff < beta, half_over_beta * diff * diff, diff - half_beta)
    out_ref[...] = jnp.sum(elem * mask, keepdims=True).reshape(1, 1, 1)


def _as_slab(a, rows_pad):
    flat = a.reshape(-1)
    need = rows_pad * _LANES - flat.shape[0]
    if need:
        flat = jnp.pad(flat, (0, need))
    return flat.reshape(rows_pad, _LANES)


def _pick_rows_tile(rows):
    for t in (2048, 1024, 512, 256, 128, 64, 32, 16, 8):
        if rows % t == 0:
            return t
    return rows


def kernel(pred, target):
    pred = pred.astype(jnp.float32)
    target = target.astype(jnp.float32)
    n = pred.shape[0]
    if n == 0:
        return jnp.float32(float("nan"))

    flat_len = n * pred.shape[1]
    rows = -(-flat_len // _LANES)
    tr = _pick_rows_tile(rows)
    rows_pad = -(-rows // tr) * tr

    p_slab = _as_slab(pred, rows_pad)
    t_slab = _as_slab(target, rows_pad)
    grid = (rows_pad // tr,)

    kernel_fn = functools.partial(
        _partials_kernel, beta=_BETA, half_over_beta=0.5 / _BETA,
        half_beta=0.5 * _BETA)
    partials = pl.pallas_call(
        kernel_fn,
        out_shape=jax.ShapeDtypeStruct((grid[0], 1, 1), jnp.float32),
        grid=grid,
        in_specs=[pl.BlockSpec((tr, _LANES), lambda i: (i, 0)),
                  pl.BlockSpec((tr, _LANES), lambda i: (i, 0))],
        out_specs=pl.BlockSpec((1, 1, 1), lambda i: (i, 0, 0)),
        compiler_params=pltpu.CompilerParams(dimension_semantics=("parallel",)),
        cost_estimate=pl.CostEstimate(
            flops=8 * rows_pad * _LANES, transcendentals=0,
            bytes_accessed=8 * rows_pad * _LANES + 4 * grid[0]),
    )(p_slab, t_slab)
    return jnp.sum(partials) * (1.0 / n)
